# baseline (speedup 1.0000x reference)
import jax
import jax.numpy as jnp
from jax import lax
from jax.experimental import pallas as pl
from jax.experimental.pallas import tpu as pltpu
from jax.experimental.pallas import tpu_sc as plsc

_DEPTH = 1000
_OUT = 64
_BATCH = 16384

_NC = 2
_NS = 16
_NW = _NC * _NS
_BPW = _BATCH // _NW
_CHUNK = 128
_NCHUNK = _BPW // _CHUNK


def _prep_body(w_ref, b_ref, t_ref):
    t_ref[...] = w_ref[...].T + b_ref[...]


def _gather_body(table_hbm, idx_hbm, out_hbm, idx_v, rows_v, sem):
    wid = lax.axis_index("s") * _NC + lax.axis_index("c")
    pltpu.sync_copy(idx_hbm.at[wid], idx_v)
    copies = [
        pltpu.async_copy(
            table_hbm.at[idx_v.at[j]],
            rows_v.at[pl.ds(j * _CHUNK, _CHUNK)],
            sem,
        )
        for j in range(_NCHUNK)
    ]
    for c in copies:
        c.wait()
    pltpu.sync_copy(rows_v, out_hbm.at[pl.ds(wid * _BPW, _BPW), 0])


def kernel(input_batch, W, b):
    idx = input_batch.astype(jnp.int32).reshape(_NW, _NCHUNK, _CHUNK)
    table = W.T + b[None, :]

    mesh = plsc.VectorSubcoreMesh(core_axis_name="c", subcore_axis_name="s")
    gather = pl.kernel(
        _gather_body,
        mesh=mesh,
        compiler_params=pltpu.CompilerParams(use_tc_tiling_on_sc=False),
        out_type=jax.ShapeDtypeStruct((_BATCH, 1, _OUT), jnp.float32),
        scratch_types=[
            pltpu.VMEM((_NCHUNK, _CHUNK), jnp.int32),
            pltpu.VMEM((_BPW, _OUT), jnp.float32),
            pltpu.SemaphoreType.DMA,
        ],
    )
    return gather(table, idx)


# speedup vs baseline: 1.4344x; 1.4344x over previous
import jax
import jax.numpy as jnp
from jax import lax
from jax.experimental import pallas as pl
from jax.experimental.pallas import tpu as pltpu
from jax.experimental.pallas import tpu_sc as plsc

_DEPTH = 1000
_OUT = 64
_LANES = 128
_BATCH = 16384

_NC = 2
_NS = 16
_NW = _NC * _NS
_BPW = _BATCH // _NW
_CHUNK = 128
_NCHUNK = _BPW // _CHUNK


def _gather_body(table_hbm, idx_hbm, out_hbm, idx_v, rows_v, sem):
    wid = lax.axis_index("s") * _NC + lax.axis_index("c")
    pltpu.sync_copy(idx_hbm.at[wid], idx_v)
    copies = [
        pltpu.async_copy(
            table_hbm.at[idx_v.at[j]],
            rows_v.at[pl.ds(j * _CHUNK, _CHUNK)],
            sem,
        )
        for j in range(_NCHUNK)
    ]
    for c in copies:
        c.wait()
    pltpu.sync_copy(rows_v, out_hbm.at[pl.ds(wid * _BPW, _BPW), 0])


def kernel(input_batch, W, b):
    idx = input_batch.astype(jnp.int32).reshape(_NW, _NCHUNK, _CHUNK)
    table = jnp.pad(W.T + b[None, :], ((0, 0), (0, _LANES - _OUT)))

    mesh = plsc.VectorSubcoreMesh(core_axis_name="c", subcore_axis_name="s")
    gather = pl.kernel(
        _gather_body,
        mesh=mesh,
        compiler_params=pltpu.CompilerParams(use_tc_tiling_on_sc=True),
        out_type=jax.ShapeDtypeStruct((_BATCH, 1, _LANES), jnp.float32),
        scratch_types=[
            pltpu.VMEM((_NCHUNK, _CHUNK), jnp.int32),
            pltpu.VMEM((_BPW, _LANES), jnp.float32),
            pltpu.SemaphoreType.DMA,
        ],
    )
    return gather(table, idx)[:, :, :_OUT]


# speedup vs baseline: 1.9006x; 1.3250x over previous
import jax
import jax.numpy as jnp
from jax import lax
from jax.experimental import pallas as pl
from jax.experimental.pallas import tpu as pltpu
from jax.experimental.pallas import tpu_sc as plsc

_DEPTH = 1000
_DEPTH_PAD = 1024
_OUT = 64
_BATCH = 16384

_NC = 2
_NS = 16
_OBANDS = 8
_BQ = 4
_ROWS = _OUT // _OBANDS
_BPT = _BATCH // _BQ
_NVEC = _BPT // 16


def _gather_body(table_hbm, idx_hbm, out_hbm, w_v, idx_v, out_v, sem):
    wid = lax.axis_index("s") * _NC + lax.axis_index("c")
    band = wid // _BQ
    quarter = lax.rem(wid, _BQ)
    pltpu.sync_copy(table_hbm.at[pl.ds(band * _ROWS, _ROWS)], w_v)
    pltpu.sync_copy(idx_hbm.at[quarter], idx_v)

    def step(k, carry):
        col = k * 16
        idx_vec = idx_v[pl.ds(col, 16)]
        for o in range(_ROWS):
            row = jnp.full((16,), o, dtype=jnp.int32)
            out_v[o, pl.ds(col, 16)] = plsc.load_gather(w_v, [row, idx_vec])
        return carry

    lax.fori_loop(0, _NVEC, step, 0)
    pltpu.sync_copy(
        out_v,
        out_hbm.at[pl.ds(band * _ROWS, _ROWS), pl.ds(quarter * _BPT, _BPT)],
    )


def kernel(input_batch, W, b):
    idx = input_batch.astype(jnp.int32).reshape(_BQ, _BPT)
    table = jnp.pad(W + b[:, None], ((0, 0), (0, _DEPTH_PAD - _DEPTH)))

    mesh = plsc.VectorSubcoreMesh(core_axis_name="c", subcore_axis_name="s")
    gather = pl.kernel(
        _gather_body,
        mesh=mesh,
        compiler_params=pltpu.CompilerParams(
            use_tc_tiling_on_sc=True, needs_layout_passes=False
        ),
        out_type=jax.ShapeDtypeStruct((_OUT, _BATCH), jnp.float32),
        scratch_types=[
            pltpu.VMEM((_ROWS, _DEPTH_PAD), jnp.float32),
            pltpu.VMEM((_BPT,), jnp.int32),
            pltpu.VMEM((_ROWS, _BPT), jnp.float32),
            pltpu.SemaphoreType.DMA,
        ],
    )
    out_t = gather(table, idx)
    return out_t.T[:, None, :]


# speedup vs baseline: 2.5226x; 1.3273x over previous
import jax
import jax.numpy as jnp
from jax import lax
from jax.experimental import pallas as pl
from jax.experimental.pallas import tpu as pltpu
from jax.experimental.pallas import tpu_sc as plsc

_DEPTH = 1000
_DEPTH_PAD = 1024
_OUT = 64
_BATCH = 16384

_NC = 2
_NS = 16
_OBANDS = 8
_BQ = 4
_ROWS = _OUT // _OBANDS
_BPT = _BATCH // _BQ
_NVEC = _BPT // 16


def _gather_body(table_hbm, idx_hbm, out_hbm, w_v, idx_v, out_v, sem):
    wid = lax.axis_index("s") * _NC + lax.axis_index("c")
    band = wid // _BQ
    quarter = lax.rem(wid, _BQ)
    pltpu.sync_copy(table_hbm.at[pl.ds(band * _ROWS, _ROWS)], w_v)
    pltpu.sync_copy(idx_hbm.at[quarter], idx_v)

    @plsc.parallel_loop(0, _BPT, step=16, unroll=4)
    def _(col):
        idx_vec = idx_v[pl.ds(col, 16)]
        for o in range(_ROWS):
            row = jnp.full((16,), o, dtype=jnp.int32)
            out_v[o, pl.ds(col, 16)] = plsc.load_gather(w_v, [row, idx_vec])
    pltpu.sync_copy(
        out_v,
        out_hbm.at[pl.ds(band * _ROWS, _ROWS), pl.ds(quarter * _BPT, _BPT)],
    )


def kernel(input_batch, W, b):
    idx = input_batch.astype(jnp.int32).reshape(_BQ, _BPT)
    table = jnp.pad(W + b[:, None], ((0, 0), (0, _DEPTH_PAD - _DEPTH)))

    mesh = plsc.VectorSubcoreMesh(core_axis_name="c", subcore_axis_name="s")
    gather = pl.kernel(
        _gather_body,
        mesh=mesh,
        compiler_params=pltpu.CompilerParams(
            use_tc_tiling_on_sc=True, needs_layout_passes=False
        ),
        out_type=jax.ShapeDtypeStruct((_OUT, _BATCH), jnp.float32),
        scratch_types=[
            pltpu.VMEM((_ROWS, _DEPTH_PAD), jnp.float32),
            pltpu.VMEM((_BPT,), jnp.int32),
            pltpu.VMEM((_ROWS, _BPT), jnp.float32),
            pltpu.SemaphoreType.DMA,
        ],
    )
    out_t = gather(table, idx)
    return out_t.T[:, None, :]


# speedup vs baseline: 2.7444x; 1.0879x over previous
import jax
import jax.numpy as jnp
from jax import lax
from jax.experimental import pallas as pl
from jax.experimental.pallas import tpu as pltpu
from jax.experimental.pallas import tpu_sc as plsc

_DEPTH = 1000
_DEPTH_PAD = 1024
_OUT = 64
_BATCH = 16384

_NC = 2
_NS = 16
_OBANDS = 8
_BQ = 4
_ROWS = _OUT // _OBANDS
_BPT = _BATCH // _BQ
_NVEC = _BPT // 16


def _gather_body(table_hbm, idx_hbm, out_hbm, w_v, idx_v, out_v, sem):
    wid = lax.axis_index("s") * _NC + lax.axis_index("c")
    band = wid // _BQ
    quarter = lax.rem(wid, _BQ)
    pltpu.sync_copy(table_hbm.at[pl.ds(band * _ROWS, _ROWS)], w_v)
    pltpu.sync_copy(idx_hbm.at[0, pl.ds(quarter * _BPT, _BPT)], idx_v)

    @plsc.parallel_loop(0, _BPT, step=16, unroll=8)
    def _(col):
        idx_vec = idx_v[pl.ds(col, 16)]
        for o in range(_ROWS):
            row = jnp.full((16,), o, dtype=jnp.int32)
            out_v[o, pl.ds(col, 16)] = plsc.load_gather(w_v, [row, idx_vec])
    pltpu.sync_copy(
        out_v,
        out_hbm.at[pl.ds(band * _ROWS, _ROWS), pl.ds(quarter * _BPT, _BPT)],
    )


def kernel(input_batch, W, b):
    idx = input_batch.astype(jnp.int32).reshape(1, _BATCH)
    table = W + b[:, None]

    mesh = plsc.VectorSubcoreMesh(core_axis_name="c", subcore_axis_name="s")
    gather = pl.kernel(
        _gather_body,
        mesh=mesh,
        compiler_params=pltpu.CompilerParams(
            use_tc_tiling_on_sc=True, needs_layout_passes=False
        ),
        out_type=jax.ShapeDtypeStruct((_OUT, _BATCH), jnp.float32),
        scratch_types=[
            pltpu.VMEM((_ROWS, _DEPTH), jnp.float32),
            pltpu.VMEM((_BPT,), jnp.int32),
            pltpu.VMEM((_ROWS, _BPT), jnp.float32),
            pltpu.SemaphoreType.DMA,
        ],
    )
    out_t = gather(table, idx)
    return out_t.T[:, None, :]
